# baseline (speedup 1.0000x reference)
import jax
import jax.numpy as jnp
from jax import lax
from jax.experimental import pallas as pl
from jax.experimental.pallas import tpu as pltpu
from jax.experimental.pallas import tpu_sc as plsc

_B = 64
_D = 256
_L = 2000
_M = 16
_N = _L * _M
_T = 0.07
_HARD_K = 50
_LAMDA = 0.5
_BLK = 16000
_NB = _N // _BLK


def _lane_sum(t_ref, v):
    acc = v[0]
    for k in range(1, 16):
        acc = acc + v[k]
    return acc


def _sc_smalls_body(feats_hbm, centers_hbm, idx_hbm, out_hbm,
                    f_v, idx_v, rows_v, orow_v, t_v, sem):
    wid = lax.axis_index("c") * 16 + lax.axis_index("s")
    iota = lax.iota(jnp.int32, 16)
    for j in range(2):
        b = wid * 2 + j
        pltpu.sync_copy(idx_hbm.at[pl.ds(b * 16, 16)], idx_v)
        pltpu.sync_copy(feats_hbm.at[pl.ds(b, 1)], f_v)
        pltpu.async_copy(centers_hbm.at[idx_v], rows_v, sem).wait()
        fch = [f_v[0, pl.ds(16 * k, 16)] for k in range(16)]
        ss = jnp.zeros((16,), jnp.float32)
        for k in range(16):
            ss = ss + fch[k] * fch[k]
        s_sc = _lane_sum(t_v, ss)
        yi = jnp.int32(0x5F3759DF) - lax.shift_right_logical(
            lax.bitcast_convert_type(s_sc, jnp.int32), 1)
        y_sc = lax.bitcast_convert_type(yi, jnp.float32)
        for _ in range(3):
            y_sc = y_sc * (1.5 - 0.5 * s_sc * y_sc * y_sc)
        y = jnp.full((16,), y_sc, jnp.float32)
        dots = jnp.zeros((16,), jnp.float32)
        for m in range(16):
            acc = jnp.zeros((16,), jnp.float32)
            for k in range(16):
                acc = acc + rows_v[m, pl.ds(16 * k, 16)] * fch[k]
            dm = _lane_sum(t_v, acc)
            dots = jnp.where(iota == m, jnp.full((16,), dm, jnp.float32), dots)
        e = jnp.exp(dots * y * (1.0 / _T))
        pos_down = _lane_sum(t_v, e)
        row = jnp.where(iota == 0, jnp.full((16,), pos_down, jnp.float32),
                        jnp.zeros((16,), jnp.float32))
        orow_v[0, pl.ds(0, 16)] = row
        pltpu.sync_copy(orow_v, out_hbm.at[pl.ds(b, 1)])


def _sc_smalls(feats, centers, idx_flat):
    mesh = plsc.VectorSubcoreMesh(core_axis_name="c", subcore_axis_name="s")
    return pl.kernel(
        _sc_smalls_body,
        mesh=mesh,
        out_type=jax.ShapeDtypeStruct((_B, 16), jnp.float32),
        scratch_types=[
            pltpu.VMEM((1, _D), jnp.float32),
            pltpu.VMEM((16,), jnp.int32),
            pltpu.VMEM((16, _D), jnp.float32),
            pltpu.VMEM((1, 16), jnp.float32),
            pltpu.VMEM((16,), jnp.float32),
            pltpu.SemaphoreType.DMA,
        ],
    )(feats, centers, idx_flat)


def _tc_dense_kernel(feats_ref, cam_ref, lab_ref, sc_ref, cen_ref, out_ref,
                     x_ref, camm_ref, acc_ref, hard_ref):
    i = pl.program_id(0)
    cam = cam_ref[...]

    @pl.when(i == 0)
    def _init():
        f = feats_ref[...]
        x_ref[...] = f / jnp.sqrt(jnp.sum(f * f, axis=1, keepdims=True))
        j = lax.broadcasted_iota(jnp.int32, (_B, _BLK), 1)
        camm_ref[...] = (jnp.bitwise_and(j, _M - 1) == cam).astype(jnp.float32)
        acc_ref[...] = jnp.zeros_like(acc_ref)
        jh = lax.broadcasted_iota(jnp.int32, (_B, 128), 1)
        lab16 = lab_ref[...] * _M
        hmask = (((jh < lab16) & (jh < _HARD_K)) |
                 ((jh >= lab16 + _M) & (jh < _HARD_K + _M)))
        hard_ref[...] = jnp.where(hmask, 1.0, 0.0)

    c = cen_ref[...]
    s = lax.dot_general(x_ref[...], c, (((1,), (1,)), ((), ())),
                        preferred_element_type=jnp.float32)
    e = jnp.exp(s * (1.0 / _T))
    ecam = e * camm_ref[...]
    fd = jnp.sum(ecam, axis=1, keepdims=True)
    j = lax.broadcasted_iota(jnp.int32, (_B, _BLK), 1)
    up_tgt = lab_ref[...] * _M + cam_ref[...] - i * _BLK
    up = jnp.sum(jnp.where(j == up_tgt, ecam, jnp.zeros_like(e)),
                 axis=1, keepdims=True)
    hscale = jnp.where(i == 0, 1.0, 0.0)
    hd = jnp.sum(e[:, :128] * hard_ref[...], axis=1, keepdims=True) * hscale
    acc_ref[...] += jnp.concatenate([fd, hd, up], axis=1)

    @pl.when(i == _NB - 1)
    def _finish():
        acc = acc_ref[...]
        sc = sc_ref[...]
        log_up = jnp.log(acc[:, 2:3])
        log_fd = jnp.log(acc[:, 0:1])
        log_pd = jnp.log(sc[:, 0:1] + acc[:, 1:2])
        intra = -jnp.sum(log_up - log_fd)
        inter = -jnp.sum(log_up - log_pd)
        out_ref[...] = jnp.concatenate(
            [intra.reshape(1, 1), (_LAMDA * inter).reshape(1, 1)], axis=1)


def kernel(feats, centers, labels, camids, epoch):
    lab32 = labels.astype(jnp.int32)
    cam32 = camids.astype(jnp.int32)
    idx_flat = (lab32[:, None] * _M +
                jnp.arange(_M, dtype=jnp.int32)[None, :]).reshape(-1)
    smalls = _sc_smalls(feats, centers, idx_flat)
    out = pl.pallas_call(
        _tc_dense_kernel,
        grid=(_NB,),
        in_specs=[
            pl.BlockSpec((_B, _D), lambda i: (0, 0)),
            pl.BlockSpec((_B, 1), lambda i: (0, 0)),
            pl.BlockSpec((_B, 1), lambda i: (0, 0)),
            pl.BlockSpec((_B, 16), lambda i: (0, 0)),
            pl.BlockSpec((_BLK, _D), lambda i: (i, 0)),
        ],
        out_specs=pl.BlockSpec((1, 2), lambda i: (0, 0)),
        out_shape=jax.ShapeDtypeStruct((1, 2), jnp.float32),
        scratch_shapes=[
            pltpu.VMEM((_B, _D), jnp.float32),
            pltpu.VMEM((_B, _BLK), jnp.float32),
            pltpu.VMEM((_B, 3), jnp.float32),
            pltpu.VMEM((_B, 128), jnp.float32),
        ],
        compiler_params=pltpu.CompilerParams(
            dimension_semantics=("arbitrary",)),
    )(feats, cam32.reshape(_B, 1), lab32.reshape(_B, 1), smalls, centers)
    gate = (jnp.asarray(epoch) >= 5).astype(jnp.float32)
    return out.reshape(2) * gate


# speedup vs baseline: 1.1725x; 1.1725x over previous
import jax
import jax.numpy as jnp
from jax import lax
from jax.experimental import pallas as pl
from jax.experimental.pallas import tpu as pltpu
from jax.experimental.pallas import tpu_sc as plsc

_B = 64
_D = 256
_L = 2000
_M = 16
_N = _L * _M
_T = 0.07
_HARD_K = 50
_LAMDA = 0.5
_BLK = 16000
_NB = _N // _BLK


def _lane_sum(t_ref, v):
    acc = v[0]
    for k in range(1, 16):
        acc = acc + v[k]
    return acc


def _sc_smalls_body(feats_hbm, centers_hbm, idx_hbm, out_hbm,
                    f_v, idx_v, rows_v, orows_v, t_v, sem, gsem):
    wid = lax.axis_index("c") * 16 + lax.axis_index("s")
    iota = lax.iota(jnp.int32, 16)
    pltpu.sync_copy(idx_hbm.at[pl.ds(wid * 32, 32)], idx_v)
    gather = pltpu.async_copy(centers_hbm.at[idx_v], rows_v, gsem)
    pltpu.sync_copy(feats_hbm.at[pl.ds(wid * 2, 2)], f_v)
    gather.wait()
    for j in range(2):
        fch = [f_v[j, pl.ds(16 * k, 16)] for k in range(16)]
        ss = jnp.zeros((16,), jnp.float32)
        for k in range(16):
            ss = ss + fch[k] * fch[k]
        s_sc = _lane_sum(t_v, ss)
        yi = jnp.int32(0x5F3759DF) - lax.shift_right_logical(
            lax.bitcast_convert_type(s_sc, jnp.int32), 1)
        y_sc = lax.bitcast_convert_type(yi, jnp.float32)
        for _ in range(3):
            y_sc = y_sc * (1.5 - 0.5 * s_sc * y_sc * y_sc)
        y = jnp.full((16,), y_sc, jnp.float32)
        dots = jnp.zeros((16,), jnp.float32)
        for m in range(16):
            acc = jnp.zeros((16,), jnp.float32)
            for k in range(16):
                acc = acc + rows_v[16 * j + m, pl.ds(16 * k, 16)] * fch[k]
            dm = _lane_sum(t_v, acc)
            dots = jnp.where(iota == m, jnp.full((16,), dm, jnp.float32), dots)
        e = jnp.exp(dots * y * (1.0 / _T))
        pos_down = _lane_sum(t_v, e)
        row = jnp.where(iota == 0, jnp.full((16,), pos_down, jnp.float32),
                        jnp.zeros((16,), jnp.float32))
        orows_v[j, pl.ds(0, 16)] = row
    pltpu.sync_copy(orows_v, out_hbm.at[pl.ds(wid * 2, 2)])


def _sc_smalls(feats, centers, idx_flat):
    mesh = plsc.VectorSubcoreMesh(core_axis_name="c", subcore_axis_name="s")
    return pl.kernel(
        _sc_smalls_body,
        mesh=mesh,
        out_type=jax.ShapeDtypeStruct((_B, 16), jnp.float32),
        scratch_types=[
            pltpu.VMEM((2, _D), jnp.float32),
            pltpu.VMEM((32,), jnp.int32),
            pltpu.VMEM((32, _D), jnp.float32),
            pltpu.VMEM((2, 16), jnp.float32),
            pltpu.VMEM((16,), jnp.float32),
            pltpu.SemaphoreType.DMA,
            pltpu.SemaphoreType.DMA,
        ],
    )(feats, centers, idx_flat)


def _tc_dense_kernel(feats_ref, cam_ref, lab_ref, cen_ref, out_ref,
                     x_ref, camm_ref, acc_ref, hard_ref):
    i = pl.program_id(0)
    cam = cam_ref[...]

    @pl.when(i == 0)
    def _init():
        f = feats_ref[...]
        x_ref[...] = f / jnp.sqrt(jnp.sum(f * f, axis=1, keepdims=True))
        j = lax.broadcasted_iota(jnp.int32, (_B, _BLK), 1)
        camm_ref[...] = (jnp.bitwise_and(j, _M - 1) == cam).astype(jnp.float32)
        acc_ref[...] = jnp.zeros_like(acc_ref)
        jh = lax.broadcasted_iota(jnp.int32, (_B, 128), 1)
        lab16 = lab_ref[...] * _M
        hmask = (((jh < lab16) & (jh < _HARD_K)) |
                 ((jh >= lab16 + _M) & (jh < _HARD_K + _M)))
        hard_ref[...] = jnp.where(hmask, 1.0, 0.0)

    c = cen_ref[...]
    s = lax.dot_general(x_ref[...], c, (((1,), (1,)), ((), ())),
                        preferred_element_type=jnp.float32)
    e = jnp.exp(s * (1.0 / _T))
    ecam = e * camm_ref[...]
    fd = jnp.sum(ecam, axis=1, keepdims=True)
    j = lax.broadcasted_iota(jnp.int32, (_B, _BLK), 1)
    up_tgt = lab_ref[...] * _M + cam_ref[...] - i * _BLK
    up = jnp.sum(jnp.where(j == up_tgt, ecam, jnp.zeros_like(e)),
                 axis=1, keepdims=True)
    hscale = jnp.where(i == 0, 1.0, 0.0)
    hd = jnp.sum(e[:, :128] * hard_ref[...], axis=1, keepdims=True) * hscale
    acc_ref[...] += jnp.concatenate([fd, hd, up], axis=1)

    @pl.when(i == _NB - 1)
    def _finish():
        acc = acc_ref[...]
        log_up = jnp.log(acc[:, 2:3])
        log_fd = jnp.log(acc[:, 0:1])
        out_ref[...] = jnp.concatenate(
            [log_up, log_fd, acc[:, 1:2], jnp.zeros((_B, 1), jnp.float32)],
            axis=1)


def _combine_kernel(tcp_ref, sc_ref, out_ref):
    tcp = tcp_ref[...]
    sc = sc_ref[...]
    log_up = tcp[:, 0:1]
    log_pd = jnp.log(sc[:, 0:1] + tcp[:, 2:3])
    intra = -jnp.sum(log_up - tcp[:, 1:2])
    inter = -jnp.sum(log_up - log_pd)
    out_ref[...] = jnp.concatenate(
        [intra.reshape(1, 1), (_LAMDA * inter).reshape(1, 1)], axis=1)


def kernel(feats, centers, labels, camids, epoch):
    lab32 = labels.astype(jnp.int32)
    cam32 = camids.astype(jnp.int32)
    idx_flat = (lab32[:, None] * _M +
                jnp.arange(_M, dtype=jnp.int32)[None, :]).reshape(-1)
    smalls = _sc_smalls(feats, centers, idx_flat)
    tc_partials = pl.pallas_call(
        _tc_dense_kernel,
        grid=(_NB,),
        in_specs=[
            pl.BlockSpec((_B, _D), lambda i: (0, 0)),
            pl.BlockSpec((_B, 1), lambda i: (0, 0)),
            pl.BlockSpec((_B, 1), lambda i: (0, 0)),
            pl.BlockSpec((_BLK, _D), lambda i: (i, 0)),
        ],
        out_specs=pl.BlockSpec((_B, 4), lambda i: (0, 0)),
        out_shape=jax.ShapeDtypeStruct((_B, 4), jnp.float32),
        scratch_shapes=[
            pltpu.VMEM((_B, _D), jnp.float32),
            pltpu.VMEM((_B, _BLK), jnp.float32),
            pltpu.VMEM((_B, 3), jnp.float32),
            pltpu.VMEM((_B, 128), jnp.float32),
        ],
        compiler_params=pltpu.CompilerParams(
            dimension_semantics=("arbitrary",)),
    )(feats, cam32.reshape(_B, 1), lab32.reshape(_B, 1), centers)
    out = pl.pallas_call(
        _combine_kernel,
        out_shape=jax.ShapeDtypeStruct((1, 2), jnp.float32),
    )(tc_partials, smalls)
    gate = (jnp.asarray(epoch) >= 5).astype(jnp.float32)
    return out.reshape(2) * gate


# speedup vs baseline: 2.2752x; 1.9406x over previous
import jax
import jax.numpy as jnp
from jax.experimental import pallas as pl
from jax.experimental.pallas import tpu as pltpu

_B = 64
_D = 256
_L = 2000
_M = 16
_N = _L * _M
_T = 0.07
_HARD_K = 50
_LAMDA = 0.5
_BLK = 16000
_NB = _N // _BLK


def _loss_kernel(feats_ref, lab_ref, cam_ref, cen_ref, out_ref,
                 x_ref, camm_ref, jdiv_ref, acc_ref, hard_ref):
    i = pl.program_id(0)
    lab = lab_ref[...]
    cam = cam_ref[...]

    @pl.when(i == 0)
    def _init():
        f = feats_ref[...]
        x_ref[...] = f / jnp.sqrt(jnp.sum(f * f, axis=1, keepdims=True))
        j = jax.lax.broadcasted_iota(jnp.int32, (_B, _BLK), 1)
        camm_ref[...] = (jnp.bitwise_and(j, _M - 1) == cam).astype(jnp.float32)
        jdiv_ref[...] = jax.lax.shift_right_logical(j, 4)
        acc_ref[...] = jnp.zeros_like(acc_ref)
        jh = jax.lax.broadcasted_iota(jnp.int32, (_B, 128), 1)
        lab16 = lab * _M
        hmask = (((jh < lab16) & (jh < _HARD_K)) |
                 ((jh >= lab16 + _M) & (jh < _HARD_K + _M)))
        hard_ref[...] = jnp.where(hmask, 1.0, 0.0)

    c = cen_ref[...]
    s = jax.lax.dot_general(x_ref[...], c, (((1,), (1,)), ((), ())),
                            preferred_element_type=jnp.float32)
    e = jnp.exp(s * (1.0 / _T))

    ecam = e * camm_ref[...]
    pos_mask = jdiv_ref[...] == (lab - i * (_BLK // _M))
    zero = jnp.zeros_like(e)
    fd = jnp.sum(ecam, axis=1, keepdims=True)
    pd = jnp.sum(jnp.where(pos_mask, e, zero), axis=1, keepdims=True)
    up = jnp.sum(jnp.where(pos_mask, ecam, zero), axis=1, keepdims=True)
    hscale = jnp.where(i == 0, 1.0, 0.0)
    hd = jnp.sum(e[:, :128] * hard_ref[...], axis=1, keepdims=True) * hscale
    acc_ref[...] += jnp.concatenate([fd, pd, up, hd], axis=1)

    @pl.when(i == _NB - 1)
    def _finish():
        acc = acc_ref[...]
        log_up = jnp.log(acc[:, 2:3])
        log_fd = jnp.log(acc[:, 0:1])
        log_pd = jnp.log(acc[:, 1:2] + acc[:, 3:4])
        intra = -jnp.sum(log_up - log_fd)
        inter = -jnp.sum(log_up - log_pd)
        out_ref[...] = jnp.concatenate(
            [intra.reshape(1, 1), (_LAMDA * inter).reshape(1, 1)], axis=1)


def kernel(feats, centers, labels, camids, epoch):
    lab = labels.reshape(_B, 1).astype(jnp.int32)
    cam = camids.reshape(_B, 1).astype(jnp.int32)
    out = pl.pallas_call(
        _loss_kernel,
        grid=(_NB,),
        in_specs=[
            pl.BlockSpec((_B, _D), lambda i: (0, 0)),
            pl.BlockSpec((_B, 1), lambda i: (0, 0)),
            pl.BlockSpec((_B, 1), lambda i: (0, 0)),
            pl.BlockSpec((_BLK, _D), lambda i: (i, 0)),
        ],
        out_specs=pl.BlockSpec((1, 2), lambda i: (0, 0)),
        out_shape=jax.ShapeDtypeStruct((1, 2), jnp.float32),
        scratch_shapes=[
            pltpu.VMEM((_B, _D), jnp.float32),
            pltpu.VMEM((_B, _BLK), jnp.float32),
            pltpu.VMEM((_B, _BLK), jnp.int32),
            pltpu.VMEM((_B, 4), jnp.float32),
            pltpu.VMEM((_B, 128), jnp.float32),
        ],
        compiler_params=pltpu.CompilerParams(
            dimension_semantics=("arbitrary",)),
    )(feats, lab, cam, centers)
    gate = (jnp.asarray(epoch) >= 5).astype(jnp.float32)
    return out.reshape(2) * gate
